# baseline (speedup 1.0000x reference)
import functools

import jax
import jax.numpy as jnp
from jax import lax
from jax.experimental import pallas as pl
from jax.experimental.pallas import tpu as pltpu
from jax.experimental.pallas import tpu_sc as plsc

B = 1024
V = 100000
NS = 16
PER = B // NS
L = 16


def _partials_body(flat_hbm, tgt_hbm, out_hbm, tgt_v, idx_v, vals_v, part_v,
                   sem):
    sid = lax.axis_index("s")
    base = sid * PER

    pltpu.sync_copy(tgt_hbm.at[pl.ds(base, PER)], tgt_v)

    lanes = lax.broadcasted_iota(jnp.int32, (L,), 0)
    for j in range(PER // L):
        row0 = base + j * L
        t = tgt_v[pl.ds(j * L, L)]
        idx_v[pl.ds(j * L, L)] = (lanes + row0) * V + t

    pltpu.async_copy(flat_hbm.at[idx_v], vals_v, sem).wait()

    acc = vals_v[pl.ds(0, L)]
    for j in range(1, PER // L):
        acc = acc + vals_v[pl.ds(j * L, L)]
    part_v[0] = acc
    pltpu.sync_copy(part_v, out_hbm.at[pl.ds(sid, 1)])


_nll_partials = functools.partial(
    pl.kernel,
    out_type=jax.ShapeDtypeStruct((NS, L), jnp.float32),
    mesh=plsc.VectorSubcoreMesh(core_axis_name="c", subcore_axis_name="s",
                                num_cores=1),
    compiler_params=pltpu.CompilerParams(needs_layout_passes=False),
    scratch_types=[
        pltpu.VMEM((PER,), jnp.int32),
        pltpu.VMEM((PER,), jnp.int32),
        pltpu.VMEM((PER,), jnp.float32),
        pltpu.VMEM((1, L), jnp.float32),
        pltpu.SemaphoreType.DMA,
    ],
)(_partials_body)


def _reduce_body(part_ref, out_ref):
    out_ref[0, 0] = jnp.sum(part_ref[...]) * (-1.0 / B)


_nll_reduce = pl.pallas_call(
    _reduce_body,
    out_shape=jax.ShapeDtypeStruct((1, 1), jnp.float32),
    in_specs=[pl.BlockSpec(memory_space=pltpu.VMEM)],
    out_specs=pl.BlockSpec(memory_space=pltpu.SMEM),
)


def kernel(inputs, targets):
    flat = inputs.reshape(-1)
    parts = _nll_partials(flat, targets.astype(jnp.int32))
    return _nll_reduce(parts)[0, 0]


# speedup vs baseline: 41.6539x; 41.6539x over previous
import functools

import jax
import jax.numpy as jnp
from jax import lax
from jax.experimental import pallas as pl
from jax.experimental.pallas import tpu as pltpu
from jax.experimental.pallas import tpu_sc as plsc

B = 1024
V = 100000
NS = 16
PER = B // NS
L = 16


def _body(at_hbm, tgt_hbm, out_hbm, tgt_v, rows_v, part_v, sem):
    sid = lax.axis_index("s")
    base = sid * PER

    pltpu.sync_copy(tgt_hbm.at[pl.ds(base, PER)], tgt_v)

    base_c = pl.multiple_of((sid // 2) * 128, 128)
    pltpu.async_copy(at_hbm.at[tgt_v, pl.ds(base_c, 128)], rows_v, sem).wait()

    lanes = lax.broadcasted_iota(jnp.int32, (L,), 0)
    col0 = (sid % 2) * PER
    acc = None
    for j in range(PER // L):
        d = lanes + j * L
        g = plsc.load_gather(rows_v, [d, d + col0])
        acc = g if acc is None else acc + g
    part_v[0] = acc
    pltpu.sync_copy(part_v, out_hbm.at[pl.ds(sid, 1)])


_partials = functools.partial(
    pl.kernel,
    out_type=jax.ShapeDtypeStruct((NS, L), jnp.float32),
    mesh=plsc.VectorSubcoreMesh(core_axis_name="c", subcore_axis_name="s",
                                num_cores=1),
    compiler_params=pltpu.CompilerParams(needs_layout_passes=False),
    scratch_types=[
        pltpu.VMEM((PER,), jnp.int32),
        pltpu.VMEM((PER, 128), jnp.float32),
        pltpu.VMEM((1, L), jnp.float32),
        pltpu.SemaphoreType.DMA,
    ],
)(_body)


def _reduce_body(part_ref, out_ref):
    out_ref[0, 0] = jnp.sum(part_ref[...]) * (-1.0 / B)


_reduce = pl.pallas_call(
    _reduce_body,
    out_shape=jax.ShapeDtypeStruct((1, 1), jnp.float32),
    in_specs=[pl.BlockSpec(memory_space=pltpu.VMEM)],
    out_specs=pl.BlockSpec(memory_space=pltpu.SMEM),
)


def kernel(inputs, targets):
    at = inputs.T
    parts = _partials(at, targets.astype(jnp.int32))
    return _reduce(parts)[0, 0]
